# baseline (speedup 1.0000x reference)
import functools

import jax
import jax.numpy as jnp
from jax import lax
from jax.experimental import pallas as pl
from jax.experimental.pallas import tpu as pltpu
from jax.experimental.pallas import tpu_sc as plsc

N = 10000
E = 320000
D = 128

NC = 2
NS = 16
NW = NC * NS
NPAD = 10240

EPW = E // NW
NVEC = EPW // 16

K = 128
CCHUNK = 80
CAP = CCHUNK * K
NR = N // NC
NRP = 5120
RPT = NRP // NS

_MESH = plsc.VectorSubcoreMesh(core_axis_name="c", subcore_axis_name="s")


@functools.partial(
    pl.kernel,
    out_type=(
        jax.ShapeDtypeStruct((NW, NPAD), jnp.float32),
        jax.ShapeDtypeStruct((NW, NC, CAP), jnp.int32),
        jax.ShapeDtypeStruct((NW, NC, CAP), jnp.int32),
        jax.ShapeDtypeStruct((NC, NW, 16), jnp.int32),
    ),
    mesh=_MESH,
    scratch_types=[
        pltpu.VMEM((EPW,), jnp.int32),
        pltpu.VMEM((EPW,), jnp.int32),
        pltpu.VMEM((NPAD,), jnp.float32),
        pltpu.VMEM((CAP,), jnp.int32),
        pltpu.VMEM((CAP,), jnp.int32),
        pltpu.VMEM((CAP,), jnp.int32),
        pltpu.VMEM((CAP,), jnp.int32),
        pltpu.VMEM((16,), jnp.int32),
    ],
    compiler_params=pltpu.CompilerParams(needs_layout_passes=False),
)
def _deg_sc(src_hbm, dst_hbm, zeros_hbm, deg_out, csrc_out, cdst_out, nch_out,
            src_v, dst_v, hist_v, cs0, cs1, cd0, cd1, cnt_v):
    c = lax.axis_index("c")
    s = lax.axis_index("s")
    wid = s * NC + c
    pltpu.sync_copy(zeros_hbm, hist_v)
    pltpu.sync_copy(src_hbm.at[wid], src_v)
    pltpu.sync_copy(dst_hbm.at[wid], dst_v)
    ones16 = jnp.ones((16,), jnp.float32)

    def body(i, carry):
        o0, o1 = carry
        s16 = src_v[pl.ds(i * 16, 16)]
        d16 = dst_v[pl.ds(i * 16, 16)]
        plsc.addupdate_scatter(hist_v, [d16], ones16)
        m0 = d16 < NR
        dloc = jnp.where(m0, d16, d16 - NR)
        plsc.store_compressed(cs0.at[pl.ds(o0, 16)], s16, mask=m0)
        plsc.store_compressed(cd0.at[pl.ds(o0, 16)], dloc, mask=m0)
        m1 = jnp.logical_not(m0)
        plsc.store_compressed(cs1.at[pl.ds(o1, 16)], s16, mask=m1)
        plsc.store_compressed(cd1.at[pl.ds(o1, 16)], dloc, mask=m1)
        pc = plsc.all_reduce_population_count(m0)[0]
        return o0 + pc, o1 + (16 - pc)

    o0, o1 = lax.fori_loop(0, NVEC, body, (jnp.int32(0), jnp.int32(0)))

    zsrc = jnp.zeros((16,), jnp.int32)
    zdst = jnp.full((16,), NRP - 1, jnp.int32)
    for t in range(K // 16):
        cs0[pl.ds(o0 + t * 16, 16)] = zsrc
        cd0[pl.ds(o0 + t * 16, 16)] = zdst
        cs1[pl.ds(o1 + t * 16, 16)] = zsrc
        cd1[pl.ds(o1 + t * 16, 16)] = zdst

    n0 = (o0 + (K - 1)) // K
    n1 = (o1 + (K - 1)) // K
    ones16i = jnp.ones((16,), jnp.int32)

    pltpu.sync_copy(hist_v, deg_out.at[wid])
    pltpu.sync_copy(cs0, csrc_out.at[wid, 0])
    pltpu.sync_copy(cs1, csrc_out.at[wid, 1])
    pltpu.sync_copy(cd0, cdst_out.at[wid, 0])
    pltpu.sync_copy(cd1, cdst_out.at[wid, 1])
    cnt_v[...] = ones16i * n0
    pltpu.sync_copy(cnt_v, nch_out.at[0, wid])
    cnt_v[...] = ones16i * n1
    pltpu.sync_copy(cnt_v, nch_out.at[1, wid])


_R = 1000


def _dinv_body(d_ref, o_ref):
    ones = jnp.ones((NW, 1), jnp.float32)
    dsum = lax.dot_general(d_ref[...], ones, (((0,), (0,)), ((), ())),
                           preferred_element_type=jnp.float32)
    o_ref[...] = lax.rsqrt(dsum + 1.0)


_dinv_tc = pl.pallas_call(
    _dinv_body,
    out_shape=jax.ShapeDtypeStruct((NPAD, 1), jnp.float32),
)


def _y_body(x_ref, w_ref, d_ref, y_ref):
    xw = jnp.dot(x_ref[...], w_ref[...], preferred_element_type=jnp.float32)
    y_ref[...] = xw * d_ref[...]


_y_tc = pl.pallas_call(
    _y_body,
    grid=(N // _R,),
    in_specs=[
        pl.BlockSpec((_R, D), lambda i: (i, 0)),
        pl.BlockSpec((D, D), lambda i: (0, 0)),
        pl.BlockSpec((_R, 1), lambda i: (i, 0)),
    ],
    out_specs=pl.BlockSpec((_R, D), lambda i: (i, 0)),
    out_shape=jax.ShapeDtypeStruct((N, D), jnp.float32),
)


@functools.partial(
    pl.kernel,
    out_type=jax.ShapeDtypeStruct((NC, NRP, D), jnp.float32),
    mesh=_MESH,
    scratch_types=[
        pltpu.VMEM((CAP,), jnp.int32),
        pltpu.VMEM((CAP,), jnp.int32),
        pltpu.VMEM((16,), jnp.int32),
        pltpu.VMEM((K, D), jnp.float32),
        pltpu.VMEM((K, D), jnp.float32),
        pltpu.VMEM_SHARED((NRP, D), jnp.float32),
        pltpu.SemaphoreType.DMA,
        pltpu.SemaphoreType.DMA,
    ],
)
def _edge_sc(y_hbm, csrc_hbm, cdst_hbm, nch_hbm, zrow_hbm, out_hbm,
             src_v, dst_v, cnt_v, buf_a, buf_b, acc_sh, sem_a, sem_b):
    c = lax.axis_index("c")
    s = lax.axis_index("s")
    for i in range(RPT // 160):
        pltpu.sync_copy(zrow_hbm, acc_sh.at[pl.ds(s * RPT + i * 160, 160)])
    plsc.subcore_barrier()

    def run_bucket(w):
        pltpu.sync_copy(csrc_hbm.at[w, c], src_v)
        pltpu.sync_copy(cdst_hbm.at[w, c], dst_v)
        pltpu.sync_copy(nch_hbm.at[c, w], cnt_v)
        n = cnt_v[pl.ds(0, 16)][0]

        @pl.when(n > 0)
        def _():
            pltpu.async_copy(y_hbm.at[src_v.at[pl.ds(0, K)]], buf_a, sem_a)

            def body(jj, carry):
                j0 = jj * 2
                j1 = j0 + 1
                s0 = pl.ds(j0 * K, K)
                s1 = pl.ds(j1 * K, K)

                @pl.when(j1 < n)
                def _():
                    pltpu.async_copy(y_hbm.at[src_v.at[s1]], buf_b, sem_b)

                pltpu.make_async_copy(y_hbm.at[src_v.at[s0]], buf_a,
                                      sem_a).wait()
                pltpu.sync_copy(buf_a, acc_sh.at[dst_v.at[s0]], add=True)

                @pl.when(j0 + 2 < n)
                def _():
                    pltpu.async_copy(y_hbm.at[src_v.at[pl.ds((j0 + 2) * K, K)]],
                                     buf_a, sem_a)

                @pl.when(j1 < n)
                def _():
                    pltpu.make_async_copy(y_hbm.at[src_v.at[s1]], buf_b,
                                          sem_b).wait()
                    pltpu.sync_copy(buf_b, acc_sh.at[dst_v.at[s1]], add=True)

                return carry

            lax.fori_loop(0, (n + 1) // 2, body, 0)

    run_bucket(s * NC)
    run_bucket(s * NC + 1)

    plsc.subcore_barrier()
    for i in range(RPT // 160):
        r = s * RPT + i * 160
        pltpu.sync_copy(acc_sh.at[pl.ds(r, 160)],
                        out_hbm.at[c, pl.ds(r, 160)])


def _fin_body(a_ref, y_ref, d_ref, b_ref, o_ref):
    o_ref[...] = d_ref[...] * (a_ref[0] + y_ref[...]) + b_ref[...]


_fin_tc = pl.pallas_call(
    _fin_body,
    grid=(N // _R,),
    in_specs=[
        pl.BlockSpec((1, _R, D), lambda i: (i // 5, i % 5, 0)),
        pl.BlockSpec((_R, D), lambda i: (i, 0)),
        pl.BlockSpec((_R, 1), lambda i: (i, 0)),
        pl.BlockSpec((1, D), lambda i: (0, 0)),
    ],
    out_specs=pl.BlockSpec((_R, D), lambda i: (i, 0)),
    out_shape=jax.ShapeDtypeStruct((N, D), jnp.float32),
)


def kernel(x, edge_index, w, bias):
    src_p = edge_index[0].reshape(NW, EPW)
    dst_p = edge_index[1].reshape(NW, EPW)
    zeros_n = jnp.zeros((NPAD,), jnp.float32)
    zeros_row = jnp.zeros((160, D), jnp.float32)

    deg, csrc, cdst, nch = _deg_sc(src_p, dst_p, zeros_n)
    dinv = _dinv_tc(deg)
    y = _y_tc(x, w, dinv)
    acc = _edge_sc(y, csrc, cdst, nch, zeros_row)
    return _fin_tc(acc, y, dinv, bias.reshape(1, D))


# speedup vs baseline: 1.1421x; 1.1421x over previous
import functools

import jax
import jax.numpy as jnp
from jax import lax
from jax.experimental import pallas as pl
from jax.experimental.pallas import tpu as pltpu
from jax.experimental.pallas import tpu_sc as plsc

N = 10000
E = 320000
D = 128

NC = 2
NS = 16
NW = NC * NS
NPAD = 10240

EPW = E // NW
NVEC = EPW // 16

K = 80
CCHUNK = 126
CAP = CCHUNK * K
NR = N // NC
NRP = 5120
RPT = NRP // NS

_MESH = plsc.VectorSubcoreMesh(core_axis_name="c", subcore_axis_name="s")


@functools.partial(
    pl.kernel,
    out_type=(
        jax.ShapeDtypeStruct((NW, NPAD), jnp.float32),
        jax.ShapeDtypeStruct((NW, NC, CAP), jnp.int32),
        jax.ShapeDtypeStruct((NW, NC, CAP), jnp.int32),
        jax.ShapeDtypeStruct((NC, NW, 16), jnp.int32),
    ),
    mesh=_MESH,
    scratch_types=[
        pltpu.VMEM((EPW,), jnp.int32),
        pltpu.VMEM((EPW,), jnp.int32),
        pltpu.VMEM((NPAD,), jnp.float32),
        pltpu.VMEM((CAP,), jnp.int32),
        pltpu.VMEM((CAP,), jnp.int32),
        pltpu.VMEM((CAP,), jnp.int32),
        pltpu.VMEM((CAP,), jnp.int32),
        pltpu.VMEM((16,), jnp.int32),
    ],
    compiler_params=pltpu.CompilerParams(needs_layout_passes=False),
)
def _deg_sc(src_hbm, dst_hbm, zeros_hbm, deg_out, csrc_out, cdst_out, nch_out,
            src_v, dst_v, hist_v, cs0, cs1, cd0, cd1, cnt_v):
    c = lax.axis_index("c")
    s = lax.axis_index("s")
    wid = s * NC + c
    pltpu.sync_copy(zeros_hbm, hist_v)
    pltpu.sync_copy(src_hbm.at[wid], src_v)
    pltpu.sync_copy(dst_hbm.at[wid], dst_v)
    ones16 = jnp.ones((16,), jnp.float32)

    def body(i, carry):
        o0, o1 = carry
        s16 = src_v[pl.ds(i * 16, 16)]
        d16 = dst_v[pl.ds(i * 16, 16)]
        plsc.addupdate_scatter(hist_v, [d16], ones16)
        m0 = d16 < NR
        dloc = jnp.where(m0, d16, d16 - NR)
        plsc.store_compressed(cs0.at[pl.ds(o0, 16)], s16, mask=m0)
        plsc.store_compressed(cd0.at[pl.ds(o0, 16)], dloc, mask=m0)
        m1 = jnp.logical_not(m0)
        plsc.store_compressed(cs1.at[pl.ds(o1, 16)], s16, mask=m1)
        plsc.store_compressed(cd1.at[pl.ds(o1, 16)], dloc, mask=m1)
        pc = plsc.all_reduce_population_count(m0)[0]
        return o0 + pc, o1 + (16 - pc)

    o0, o1 = lax.fori_loop(0, NVEC, body, (jnp.int32(0), jnp.int32(0)))

    zsrc = jnp.zeros((16,), jnp.int32)
    zdst = jnp.full((16,), NRP - 1, jnp.int32)
    for t in range(K // 16):
        cs0[pl.ds(o0 + t * 16, 16)] = zsrc
        cd0[pl.ds(o0 + t * 16, 16)] = zdst
        cs1[pl.ds(o1 + t * 16, 16)] = zsrc
        cd1[pl.ds(o1 + t * 16, 16)] = zdst

    n0 = (o0 + (K - 1)) // K
    n1 = (o1 + (K - 1)) // K
    ones16i = jnp.ones((16,), jnp.int32)

    pltpu.sync_copy(hist_v, deg_out.at[wid])
    pltpu.sync_copy(cs0, csrc_out.at[wid, 0])
    pltpu.sync_copy(cs1, csrc_out.at[wid, 1])
    pltpu.sync_copy(cd0, cdst_out.at[wid, 0])
    pltpu.sync_copy(cd1, cdst_out.at[wid, 1])
    cnt_v[...] = ones16i * n0
    pltpu.sync_copy(cnt_v, nch_out.at[0, wid])
    cnt_v[...] = ones16i * n1
    pltpu.sync_copy(cnt_v, nch_out.at[1, wid])


_R = 1000


def _dinv_body(d_ref, o_ref):
    ones = jnp.ones((NW, 1), jnp.float32)
    dsum = lax.dot_general(d_ref[...], ones, (((0,), (0,)), ((), ())),
                           preferred_element_type=jnp.float32)
    o_ref[...] = lax.rsqrt(dsum + 1.0)


_dinv_tc = pl.pallas_call(
    _dinv_body,
    out_shape=jax.ShapeDtypeStruct((NPAD, 1), jnp.float32),
)


def _y_body(x_ref, w_ref, d_ref, y_ref):
    xw = jnp.dot(x_ref[...], w_ref[...], preferred_element_type=jnp.float32)
    y_ref[...] = xw * d_ref[...]


_y_tc = pl.pallas_call(
    _y_body,
    grid=(N // _R,),
    in_specs=[
        pl.BlockSpec((_R, D), lambda i: (i, 0)),
        pl.BlockSpec((D, D), lambda i: (0, 0)),
        pl.BlockSpec((_R, 1), lambda i: (i, 0)),
    ],
    out_specs=pl.BlockSpec((_R, D), lambda i: (i, 0)),
    out_shape=jax.ShapeDtypeStruct((N, D), jnp.float32),
)


@functools.partial(
    pl.kernel,
    out_type=jax.ShapeDtypeStruct((NC, NRP, D), jnp.float32),
    mesh=_MESH,
    scratch_types=[
        pltpu.VMEM((CAP,), jnp.int32),
        pltpu.VMEM((CAP,), jnp.int32),
        pltpu.VMEM((16,), jnp.int32),
        pltpu.VMEM((K, D), jnp.float32),
        pltpu.VMEM((K, D), jnp.float32),
        pltpu.VMEM_SHARED((NRP, D), jnp.float32),
        pltpu.SemaphoreType.DMA,
        pltpu.SemaphoreType.DMA,
    ],
)
def _edge_sc(y_hbm, csrc_hbm, cdst_hbm, nch_hbm, zrow_hbm, out_hbm,
             src_v, dst_v, cnt_v, buf_a, buf_b, acc_sh, sem_a, sem_b):
    c = lax.axis_index("c")
    s = lax.axis_index("s")
    for i in range(RPT // 160):
        pltpu.sync_copy(zrow_hbm, acc_sh.at[pl.ds(s * RPT + i * 160, 160)])
    plsc.subcore_barrier()

    def run_bucket(w):
        pltpu.sync_copy(csrc_hbm.at[w, c], src_v)
        pltpu.sync_copy(cdst_hbm.at[w, c], dst_v)
        pltpu.sync_copy(nch_hbm.at[c, w], cnt_v)
        n = cnt_v[pl.ds(0, 16)][0]

        @pl.when(n > 0)
        def _():
            pltpu.async_copy(y_hbm.at[src_v.at[pl.ds(0, K)]], buf_a, sem_a)

            def body(jj, carry):
                j0 = jj * 2
                j1 = j0 + 1
                s0 = pl.ds(j0 * K, K)
                s1 = pl.ds(j1 * K, K)

                @pl.when(j1 < n)
                def _():
                    pltpu.async_copy(y_hbm.at[src_v.at[s1]], buf_b, sem_b)

                pltpu.make_async_copy(y_hbm.at[src_v.at[s0]], buf_a,
                                      sem_a).wait()
                pltpu.sync_copy(buf_a, acc_sh.at[dst_v.at[s0]], add=True)

                @pl.when(j0 + 2 < n)
                def _():
                    pltpu.async_copy(y_hbm.at[src_v.at[pl.ds((j0 + 2) * K, K)]],
                                     buf_a, sem_a)

                @pl.when(j1 < n)
                def _():
                    pltpu.make_async_copy(y_hbm.at[src_v.at[s1]], buf_b,
                                          sem_b).wait()
                    pltpu.sync_copy(buf_b, acc_sh.at[dst_v.at[s1]], add=True)

                return carry

            lax.fori_loop(0, (n + 1) // 2, body, 0)

    run_bucket(s * NC)
    run_bucket(s * NC + 1)

    plsc.subcore_barrier()
    for i in range(RPT // 160):
        r = s * RPT + i * 160
        pltpu.sync_copy(acc_sh.at[pl.ds(r, 160)],
                        out_hbm.at[c, pl.ds(r, 160)])


def _fin_body(a_ref, y_ref, d_ref, b_ref, o_ref):
    o_ref[...] = d_ref[...] * (a_ref[0] + y_ref[...]) + b_ref[...]


_fin_tc = pl.pallas_call(
    _fin_body,
    grid=(N // _R,),
    in_specs=[
        pl.BlockSpec((1, _R, D), lambda i: (i // 5, i % 5, 0)),
        pl.BlockSpec((_R, D), lambda i: (i, 0)),
        pl.BlockSpec((_R, 1), lambda i: (i, 0)),
        pl.BlockSpec((1, D), lambda i: (0, 0)),
    ],
    out_specs=pl.BlockSpec((_R, D), lambda i: (i, 0)),
    out_shape=jax.ShapeDtypeStruct((N, D), jnp.float32),
)


def kernel(x, edge_index, w, bias):
    src_p = edge_index[0].reshape(NW, EPW)
    dst_p = edge_index[1].reshape(NW, EPW)
    zeros_n = jnp.zeros((NPAD,), jnp.float32)
    zeros_row = jnp.zeros((160, D), jnp.float32)

    deg, csrc, cdst, nch = _deg_sc(src_p, dst_p, zeros_n)
    dinv = _dinv_tc(deg)
    y = _y_tc(x, w, dinv)
    acc = _edge_sc(y, csrc, cdst, nch, zeros_row)
    return _fin_tc(acc, y, dinv, bias.reshape(1, D))


# speedup vs baseline: 1.1449x; 1.0025x over previous
import functools

import jax
import jax.numpy as jnp
from jax import lax
from jax.experimental import pallas as pl
from jax.experimental.pallas import tpu as pltpu
from jax.experimental.pallas import tpu_sc as plsc

N = 10000
E = 320000
D = 128

NC = 2
NS = 16
NW = NC * NS
NPAD = 10240

EPW = E // NW
NVEC = EPW // 16

K = 80
CCHUNK = 126
CAP = CCHUNK * K
NR = N // NC
NRP = 5120
RPT = NRP // NS

_MESH = plsc.VectorSubcoreMesh(core_axis_name="c", subcore_axis_name="s")


@functools.partial(
    pl.kernel,
    out_type=(
        jax.ShapeDtypeStruct((NW, NPAD), jnp.float32),
        jax.ShapeDtypeStruct((NW, NC, CAP), jnp.int32),
        jax.ShapeDtypeStruct((NW, NC, CAP), jnp.int32),
        jax.ShapeDtypeStruct((NC, NW, 16), jnp.int32),
    ),
    mesh=_MESH,
    scratch_types=[
        pltpu.VMEM((EPW,), jnp.int32),
        pltpu.VMEM((EPW,), jnp.int32),
        pltpu.VMEM((NPAD,), jnp.float32),
        pltpu.VMEM((CAP,), jnp.int32),
        pltpu.VMEM((CAP,), jnp.int32),
        pltpu.VMEM((CAP,), jnp.int32),
        pltpu.VMEM((CAP,), jnp.int32),
        pltpu.VMEM((16,), jnp.int32),
    ],
    compiler_params=pltpu.CompilerParams(needs_layout_passes=False),
)
def _deg_sc(src_hbm, dst_hbm, zeros_hbm, deg_out, csrc_out, cdst_out, nch_out,
            src_v, dst_v, hist_v, cs0, cs1, cd0, cd1, cnt_v):
    c = lax.axis_index("c")
    s = lax.axis_index("s")
    wid = s * NC + c
    pltpu.sync_copy(zeros_hbm, hist_v)
    pltpu.sync_copy(src_hbm.at[wid], src_v)
    pltpu.sync_copy(dst_hbm.at[wid], dst_v)
    ones16 = jnp.ones((16,), jnp.float32)

    def body(i, carry):
        o0, o1 = carry
        s16 = src_v[pl.ds(i * 16, 16)]
        d16 = dst_v[pl.ds(i * 16, 16)]
        plsc.addupdate_scatter(hist_v, [d16], ones16)
        m0 = d16 < NR
        dloc = jnp.where(m0, d16, d16 - NR)
        plsc.store_compressed(cs0.at[pl.ds(o0, 16)], s16, mask=m0)
        plsc.store_compressed(cd0.at[pl.ds(o0, 16)], dloc, mask=m0)
        m1 = jnp.logical_not(m0)
        plsc.store_compressed(cs1.at[pl.ds(o1, 16)], s16, mask=m1)
        plsc.store_compressed(cd1.at[pl.ds(o1, 16)], dloc, mask=m1)
        pc = plsc.all_reduce_population_count(m0)[0]
        return o0 + pc, o1 + (16 - pc)

    o0, o1 = lax.fori_loop(0, NVEC, body, (jnp.int32(0), jnp.int32(0)))

    zsrc = jnp.zeros((16,), jnp.int32)
    zdst = jnp.full((16,), NRP - 1, jnp.int32)
    for t in range(K // 16):
        cs0[pl.ds(o0 + t * 16, 16)] = zsrc
        cd0[pl.ds(o0 + t * 16, 16)] = zdst
        cs1[pl.ds(o1 + t * 16, 16)] = zsrc
        cd1[pl.ds(o1 + t * 16, 16)] = zdst

    n0 = (o0 + (K - 1)) // K
    n1 = (o1 + (K - 1)) // K
    ones16i = jnp.ones((16,), jnp.int32)

    pltpu.sync_copy(hist_v, deg_out.at[wid])
    pltpu.sync_copy(cs0, csrc_out.at[wid, 0])
    pltpu.sync_copy(cs1, csrc_out.at[wid, 1])
    pltpu.sync_copy(cd0, cdst_out.at[wid, 0])
    pltpu.sync_copy(cd1, cdst_out.at[wid, 1])
    cnt_v[...] = ones16i * n0
    pltpu.sync_copy(cnt_v, nch_out.at[0, wid])
    cnt_v[...] = ones16i * n1
    pltpu.sync_copy(cnt_v, nch_out.at[1, wid])


_R = 1000


def _dinv_body(d_ref, o_ref):
    ones = jnp.ones((NW, 1), jnp.float32)
    dsum = lax.dot_general(d_ref[...], ones, (((0,), (0,)), ((), ())),
                           preferred_element_type=jnp.float32)
    o_ref[...] = lax.rsqrt(dsum + 1.0)


_dinv_tc = pl.pallas_call(
    _dinv_body,
    out_shape=jax.ShapeDtypeStruct((NPAD, 1), jnp.float32),
)


def _y_body(x_ref, w_ref, d_ref, y_ref):
    xw = jnp.dot(x_ref[...], w_ref[...], preferred_element_type=jnp.float32)
    y_ref[...] = xw * d_ref[...]


_y_tc = pl.pallas_call(
    _y_body,
    grid=(N // _R,),
    in_specs=[
        pl.BlockSpec((_R, D), lambda i: (i, 0)),
        pl.BlockSpec((D, D), lambda i: (0, 0)),
        pl.BlockSpec((_R, 1), lambda i: (i, 0)),
    ],
    out_specs=pl.BlockSpec((_R, D), lambda i: (i, 0)),
    out_shape=jax.ShapeDtypeStruct((N, D), jnp.float32),
)


@functools.partial(
    pl.kernel,
    out_type=jax.ShapeDtypeStruct((NC, NRP, D), jnp.float32),
    mesh=_MESH,
    scratch_types=[
        pltpu.VMEM((CAP,), jnp.int32),
        pltpu.VMEM((CAP,), jnp.int32),
        pltpu.VMEM((16,), jnp.int32),
        pltpu.VMEM((K, D), jnp.float32),
        pltpu.VMEM((K, D), jnp.float32),
        pltpu.VMEM_SHARED((NRP, D), jnp.float32),
        pltpu.SemaphoreType.DMA,
        pltpu.SemaphoreType.DMA,
    ],
)
def _edge_sc(y_hbm, csrc_hbm, cdst_hbm, nch_hbm, zrow_hbm, out_hbm,
             src_v, dst_v, cnt_v, buf_a, buf_b, acc_sh, sem_a, sem_b):
    c = lax.axis_index("c")
    s = lax.axis_index("s")
    for i in range(RPT // 160):
        pltpu.sync_copy(zrow_hbm, acc_sh.at[pl.ds(s * RPT + i * 160, 160)])
    plsc.subcore_barrier()

    def run_bucket(w):
        pltpu.sync_copy(csrc_hbm.at[w, c], src_v)
        pltpu.sync_copy(cdst_hbm.at[w, c], dst_v)
        pltpu.sync_copy(nch_hbm.at[c, w], cnt_v)
        n = cnt_v[pl.ds(0, 16)][0]

        @pl.when(n > 0)
        def _():
            pltpu.async_copy(y_hbm.at[src_v.at[pl.ds(0, K)]], buf_a, sem_a)

            def body(jj, carry):
                j0 = jj * 2
                j1 = j0 + 1
                s0 = pl.ds(j0 * K, K)
                s1 = pl.ds(j1 * K, K)
                pltpu.async_copy(y_hbm.at[src_v.at[s1]], buf_b, sem_b)
                pltpu.make_async_copy(y_hbm.at[src_v.at[s0]], buf_a,
                                      sem_a).wait()
                pltpu.sync_copy(buf_a, acc_sh.at[dst_v.at[s0]], add=True)
                pltpu.async_copy(y_hbm.at[src_v.at[pl.ds((j0 + 2) * K, K)]],
                                 buf_a, sem_a)
                pltpu.make_async_copy(y_hbm.at[src_v.at[s1]], buf_b,
                                      sem_b).wait()
                pltpu.sync_copy(buf_b, acc_sh.at[dst_v.at[s1]], add=True)
                return carry

            lax.fori_loop(0, (n - 1) // 2, body, 0)

            m = n - 2 * ((n - 1) // 2)
            jt = n - m
            st = pl.ds(jt * K, K)
            sl = pl.ds((n - 1) * K, K)

            @pl.when(m == 2)
            def _():
                pltpu.async_copy(y_hbm.at[src_v.at[sl]], buf_b, sem_b)

            pltpu.make_async_copy(y_hbm.at[src_v.at[st]], buf_a, sem_a).wait()
            pltpu.sync_copy(buf_a, acc_sh.at[dst_v.at[st]], add=True)

            @pl.when(m == 2)
            def _():
                pltpu.make_async_copy(y_hbm.at[src_v.at[sl]], buf_b,
                                      sem_b).wait()
                pltpu.sync_copy(buf_b, acc_sh.at[dst_v.at[sl]], add=True)

    run_bucket(s * NC)
    run_bucket(s * NC + 1)

    plsc.subcore_barrier()
    for i in range(RPT // 160):
        r = s * RPT + i * 160
        pltpu.sync_copy(acc_sh.at[pl.ds(r, 160)],
                        out_hbm.at[c, pl.ds(r, 160)])


def _fin_body(a_ref, y_ref, d_ref, b_ref, o_ref):
    o_ref[...] = d_ref[...] * (a_ref[0] + y_ref[...]) + b_ref[...]


_fin_tc = pl.pallas_call(
    _fin_body,
    grid=(N // _R,),
    in_specs=[
        pl.BlockSpec((1, _R, D), lambda i: (i // 5, i % 5, 0)),
        pl.BlockSpec((_R, D), lambda i: (i, 0)),
        pl.BlockSpec((_R, 1), lambda i: (i, 0)),
        pl.BlockSpec((1, D), lambda i: (0, 0)),
    ],
    out_specs=pl.BlockSpec((_R, D), lambda i: (i, 0)),
    out_shape=jax.ShapeDtypeStruct((N, D), jnp.float32),
)


def kernel(x, edge_index, w, bias):
    src_p = edge_index[0].reshape(NW, EPW)
    dst_p = edge_index[1].reshape(NW, EPW)
    zeros_n = jnp.zeros((NPAD,), jnp.float32)
    zeros_row = jnp.zeros((160, D), jnp.float32)

    deg, csrc, cdst, nch = _deg_sc(src_p, dst_p, zeros_n)
    dinv = _dinv_tc(deg)
    y = _y_tc(x, w, dinv)
    acc = _edge_sc(y, csrc, cdst, nch, zeros_row)
    return _fin_tc(acc, y, dinv, bias.reshape(1, D))
